# baseline (speedup 1.0000x reference)
import functools

import jax
import jax.numpy as jnp
from jax import lax
from jax.experimental import pallas as pl
from jax.experimental.pallas import tpu as pltpu
from jax.experimental.pallas import tpu_sc as plsc

N = 10000
D = 256
DH = 128
E = 160000
NUM_LAYERS = 5
NC = 2
NS = 16

CHUNK = 128
EP = 163840
EPS = EP // NS
NCHUNK = EPS // CHUNK
NP = 10240
RPS = NP // NS
ZC = 64

EPC = E // NC
DPS = EPC // NS
DCHUNK = 40
DNCHUNK = DPS // DCHUNK
CW = 128

BN = 1000

_MESH = plsc.VectorSubcoreMesh(core_axis_name="c", subcore_axis_name="s")


def _deg_counts(dst):
    ones = jnp.ones((DCHUNK, CW), jnp.float32)
    zeros = jnp.zeros((RPS, CW), jnp.float32)

    @functools.partial(
        pl.kernel,
        mesh=_MESH,
        out_type=jax.ShapeDtypeStruct((NC, NP, CW), jnp.float32),
        scratch_types=[
            pltpu.VMEM((DCHUNK,), jnp.int32),
            pltpu.VMEM((DCHUNK, CW), jnp.float32),
            pltpu.VMEM_SHARED((NP, CW), jnp.float32),
            pltpu.SemaphoreType.DMA,
        ],
    )
    def k(dst_hbm, ones_hbm, zeros_hbm, cnt_hbm, idx_v, ones_v, acc_sh, sem):
        c = lax.axis_index("c")
        s = lax.axis_index("s")
        row = pl.ds(s * RPS, RPS)
        pltpu.sync_copy(zeros_hbm, acc_sh.at[row])
        pltpu.sync_copy(ones_hbm, ones_v)
        plsc.subcore_barrier()
        base = c * EPC + s * DPS

        @pl.loop(0, DNCHUNK)
        def _(g):
            pltpu.sync_copy(dst_hbm.at[pl.ds(base + g * DCHUNK, DCHUNK)],
                            idx_v)
            pltpu.sync_copy(ones_v, acc_sh.at[idx_v], add=True)

        plsc.subcore_barrier()
        pltpu.sync_copy(acc_sh.at[row], cnt_hbm.at[c, row])

    return k(dst, ones, zeros)


def _aggregate(h2s, sidx2, dst2):
    zeros = jnp.zeros((RPS, DH), jnp.float32)

    @functools.partial(
        pl.kernel,
        mesh=_MESH,
        out_type=jax.ShapeDtypeStruct((NC, NP, DH), jnp.float32),
        scratch_types=[
            pltpu.VMEM((NCHUNK, CHUNK), jnp.int32),
            pltpu.VMEM((1, CHUNK), jnp.int32),
            pltpu.VMEM((1, CHUNK), jnp.int32),
            pltpu.VMEM((CHUNK, DH), jnp.float32),
            pltpu.VMEM((CHUNK, DH), jnp.float32),
            pltpu.VMEM_SHARED((NP, DH), jnp.float32),
            pltpu.SemaphoreType.DMA,
            pltpu.SemaphoreType.DMA,
        ],
    )
    def k(h2s_hbm, sidx_hbm, dst_hbm, zeros_hbm, out_hbm,
          src_slab, dst_v0, dst_v1, rows0, rows1, acc_sh, sem0, sem1):
        c = lax.axis_index("c")
        s = lax.axis_index("s")
        row = pl.ds(s * RPS, RPS)
        pltpu.sync_copy(sidx_hbm.at[pl.ds((c * NS + s) * NCHUNK, NCHUNK)],
                        src_slab)
        pltpu.sync_copy(zeros_hbm, acc_sh.at[row])
        plsc.subcore_barrier()

        dbase = s * NCHUNK

        @pl.loop(0, NCHUNK, step=2)
        def _(g):
            cp0 = pltpu.async_copy(h2s_hbm.at[src_slab.at[g]], rows0, sem0)
            cp1 = pltpu.async_copy(h2s_hbm.at[src_slab.at[g + 1]], rows1,
                                   sem1)
            pltpu.sync_copy(dst_hbm.at[pl.ds(dbase + g, 1)], dst_v0)
            pltpu.sync_copy(dst_hbm.at[pl.ds(dbase + g + 1, 1)], dst_v1)
            cp0.wait()
            pltpu.sync_copy(rows0, acc_sh.at[dst_v0.at[0]], add=True)
            cp1.wait()
            pltpu.sync_copy(rows1, acc_sh.at[dst_v1.at[0]], add=True)

        plsc.subcore_barrier()
        pltpu.sync_copy(acc_sh.at[row], out_hbm.at[c, row])

    return k(h2s, sidx2, dst2, zeros)


def _dinv_from_counts(cnt):
    def body(c_r, o):
        deg = c_r[0] + c_r[1] + 1.0
        o[...] = lax.rsqrt(deg)

    return pl.pallas_call(
        body,
        grid=(N // BN,),
        in_specs=[pl.BlockSpec((NC, BN, CW), lambda i: (0, i, 0))],
        out_specs=pl.BlockSpec((BN, CW), lambda i: (i, 0)),
        out_shape=jax.ShapeDtypeStruct((N, CW), jnp.float32),
    )(cnt)


def _matmul_first(x, W, dinv):
    def body(x_r, w_r, dv_r, o):
        h = lax.dot_general(x_r[...], w_r[...], (((1,), (0,)), ((), ())),
                            precision=lax.Precision.HIGHEST,
                            preferred_element_type=jnp.float32)
        dv = dv_r[...]
        o[0] = h[:, :DH] * dv
        o[1] = h[:, DH:] * dv

    return pl.pallas_call(
        body,
        grid=(N // BN,),
        in_specs=[pl.BlockSpec((BN, D), lambda i: (i, 0)),
                  pl.BlockSpec((D, D), lambda i: (0, 0)),
                  pl.BlockSpec((BN, CW), lambda i: (i, 0))],
        out_specs=pl.BlockSpec((NC, BN, DH), lambda i: (0, i, 0)),
        out_shape=jax.ShapeDtypeStruct((NC, N, DH), jnp.float32),
    )(x, W, dinv)


def _matmul_mid(acc, h2, dinv, b_prev, W):
    def body(a_r, h_r, dv_r, b_r, w_r, o):
        dv = dv_r[...]
        za = dv * (a_r[0] + h_r[0]) + b_r[:, :DH]
        zb = dv * (a_r[1] + h_r[1]) + b_r[:, DH:]
        z = jnp.maximum(jnp.concatenate([za, zb], axis=1), 0.0)
        h = lax.dot_general(z, w_r[...], (((1,), (0,)), ((), ())),
                            precision=lax.Precision.HIGHEST,
                            preferred_element_type=jnp.float32)
        o[0] = h[:, :DH] * dv
        o[1] = h[:, DH:] * dv

    return pl.pallas_call(
        body,
        grid=(N // BN,),
        in_specs=[pl.BlockSpec((NC, BN, DH), lambda i: (0, i, 0)),
                  pl.BlockSpec((NC, BN, DH), lambda i: (0, i, 0)),
                  pl.BlockSpec((BN, CW), lambda i: (i, 0)),
                  pl.BlockSpec((1, D), lambda i: (0, 0)),
                  pl.BlockSpec((D, D), lambda i: (0, 0))],
        out_specs=pl.BlockSpec((NC, BN, DH), lambda i: (0, i, 0)),
        out_shape=jax.ShapeDtypeStruct((NC, N, DH), jnp.float32),
    )(acc, h2, dinv, b_prev, W)


def _epilogue_last(acc, h2, dinv, b):
    def body(a_r, h_r, dv_r, b_r, o):
        dv = dv_r[...]
        oa = dv * (a_r[0] + h_r[0]) + b_r[:, :DH]
        ob = dv * (a_r[1] + h_r[1]) + b_r[:, DH:]
        o[...] = jnp.concatenate([oa, ob], axis=1)

    return pl.pallas_call(
        body,
        grid=(N // BN,),
        in_specs=[pl.BlockSpec((NC, BN, DH), lambda i: (0, i, 0)),
                  pl.BlockSpec((NC, BN, DH), lambda i: (0, i, 0)),
                  pl.BlockSpec((BN, CW), lambda i: (i, 0)),
                  pl.BlockSpec((1, D), lambda i: (0, 0))],
        out_specs=pl.BlockSpec((BN, D), lambda i: (i, 0)),
        out_shape=jax.ShapeDtypeStruct((N, D), jnp.float32),
    )(acc, h2, dinv, b)


def kernel(x, edge_index, W0, b0, W1, b1, W2, b2, W3, b3, W4, b4):
    src = edge_index[0]
    dst = edge_index[1]
    pad = EP - E
    src_p = jnp.concatenate([src, jnp.zeros((pad,), jnp.int32)])
    sidx2 = jnp.concatenate([src_p, src_p + N]).reshape(2 * EP // CHUNK, CHUNK)
    dst2 = jnp.concatenate(
        [dst, jnp.full((pad,), N, jnp.int32)]).reshape(EP // CHUNK, CHUNK)
    Ws = [W0, W1, W2, W3, W4]
    bs = [b.reshape(1, D) for b in (b0, b1, b2, b3, b4)]

    cnt = _deg_counts(dst)
    dinv = _dinv_from_counts(cnt)

    h2 = _matmul_first(x, Ws[0], dinv)
    for i in range(1, NUM_LAYERS):
        acc = _aggregate(h2.reshape(NC * N, DH), sidx2, dst2)
        h2 = _matmul_mid(acc, h2, dinv, bs[i - 1], Ws[i])
    acc = _aggregate(h2.reshape(NC * N, DH), sidx2, dst2)
    return _epilogue_last(acc, h2, dinv, bs[-1])


# speedup vs baseline: 1.0875x; 1.0875x over previous
import functools

import jax
import jax.numpy as jnp
from jax import lax
from jax.experimental import pallas as pl
from jax.experimental.pallas import tpu as pltpu
from jax.experimental.pallas import tpu_sc as plsc

N = 10000
D = 256
DH = 128
E = 160000
NUM_LAYERS = 5
NC = 2
NS = 16

CHUNK = 80
EP = E
EPS = EP // NS
NCHUNK = EPS // CHUNK
NP = 10240
RPS = NP // NS
ZC = 64

EPC = E // NC
DPS = EPC // NS
DCHUNK = 40
DNCHUNK = DPS // DCHUNK
CW = 128

BN = 1000

_MESH = plsc.VectorSubcoreMesh(core_axis_name="c", subcore_axis_name="s")


def _deg_counts(dst):
    ones = jnp.ones((DCHUNK, CW), jnp.float32)
    zeros = jnp.zeros((RPS, CW), jnp.float32)

    @functools.partial(
        pl.kernel,
        mesh=_MESH,
        out_type=jax.ShapeDtypeStruct((NC, NP, CW), jnp.float32),
        scratch_types=[
            pltpu.VMEM((DCHUNK,), jnp.int32),
            pltpu.VMEM((DCHUNK, CW), jnp.float32),
            pltpu.VMEM_SHARED((NP, CW), jnp.float32),
            pltpu.SemaphoreType.DMA,
        ],
    )
    def k(dst_hbm, ones_hbm, zeros_hbm, cnt_hbm, idx_v, ones_v, acc_sh, sem):
        c = lax.axis_index("c")
        s = lax.axis_index("s")
        row = pl.ds(s * RPS, RPS)
        pltpu.sync_copy(zeros_hbm, acc_sh.at[row])
        pltpu.sync_copy(ones_hbm, ones_v)
        plsc.subcore_barrier()
        base = c * EPC + s * DPS

        @pl.loop(0, DNCHUNK)
        def _(g):
            pltpu.sync_copy(dst_hbm.at[pl.ds(base + g * DCHUNK, DCHUNK)],
                            idx_v)
            pltpu.sync_copy(ones_v, acc_sh.at[idx_v], add=True)

        plsc.subcore_barrier()
        pltpu.sync_copy(acc_sh.at[row], cnt_hbm.at[c, row])

    return k(dst, ones, zeros)


def _aggregate(h2s, sidx, dst):
    zeros = jnp.zeros((RPS, DH), jnp.float32)

    @functools.partial(
        pl.kernel,
        mesh=_MESH,
        out_type=jax.ShapeDtypeStruct((NC, NP, DH), jnp.float32),
        scratch_types=[
            pltpu.VMEM((CHUNK,), jnp.int32),
            pltpu.VMEM((CHUNK,), jnp.int32),
            pltpu.VMEM((CHUNK, DH), jnp.float32),
            pltpu.VMEM_SHARED((NP, DH), jnp.float32),
            pltpu.SemaphoreType.DMA,
        ],
    )
    def k(h2s_hbm, sidx_hbm, dst_hbm, zeros_hbm, out_hbm,
          src_v, dst_v, rows_v, acc_sh, sem):
        c = lax.axis_index("c")
        s = lax.axis_index("s")
        row = pl.ds(s * RPS, RPS)
        pltpu.sync_copy(zeros_hbm, acc_sh.at[row])
        plsc.subcore_barrier()
        base = c * E + s * EPS

        @pl.loop(0, NCHUNK)
        def _(g):
            off = base + g * CHUNK
            pltpu.sync_copy(sidx_hbm.at[pl.ds(off, CHUNK)], src_v)
            pltpu.sync_copy(dst_hbm.at[pl.ds(s * EPS + g * CHUNK, CHUNK)],
                            dst_v)
            pltpu.async_copy(h2s_hbm.at[src_v], rows_v, sem).wait()
            pltpu.sync_copy(rows_v, acc_sh.at[dst_v], add=True)

        plsc.subcore_barrier()
        pltpu.sync_copy(acc_sh.at[row], out_hbm.at[c, row])

    return k(h2s, sidx, dst, zeros)


def _matmul_first(x, W, cnt):
    def body(x_r, w_r, c_r, o, dv_o):
        h = lax.dot_general(x_r[...], w_r[...], (((1,), (0,)), ((), ())),
                            precision=lax.Precision.HIGHEST,
                            preferred_element_type=jnp.float32)
        dv = lax.rsqrt(c_r[0] + c_r[1] + 1.0)
        dv_o[...] = dv
        o[0] = h[:, :DH] * dv
        o[1] = h[:, DH:] * dv

    return pl.pallas_call(
        body,
        grid=(N // BN,),
        in_specs=[pl.BlockSpec((BN, D), lambda i: (i, 0)),
                  pl.BlockSpec((D, D), lambda i: (0, 0)),
                  pl.BlockSpec((NC, BN, CW), lambda i: (0, i, 0))],
        out_specs=[pl.BlockSpec((NC, BN, DH), lambda i: (0, i, 0)),
                   pl.BlockSpec((BN, CW), lambda i: (i, 0))],
        out_shape=[jax.ShapeDtypeStruct((NC, N, DH), jnp.float32),
                   jax.ShapeDtypeStruct((N, CW), jnp.float32)],
    )(x, W, cnt)


def _matmul_mid(acc, h2, dinv, b_prev, W):
    def body(a_r, h_r, dv_r, b_r, w_r, o):
        dv = dv_r[...]
        za = dv * (a_r[0] + h_r[0]) + b_r[:, :DH]
        zb = dv * (a_r[1] + h_r[1]) + b_r[:, DH:]
        z = jnp.maximum(jnp.concatenate([za, zb], axis=1), 0.0)
        h = lax.dot_general(z, w_r[...], (((1,), (0,)), ((), ())),
                            precision=lax.Precision.HIGHEST,
                            preferred_element_type=jnp.float32)
        o[0] = h[:, :DH] * dv
        o[1] = h[:, DH:] * dv

    return pl.pallas_call(
        body,
        grid=(N // BN,),
        in_specs=[pl.BlockSpec((NC, BN, DH), lambda i: (0, i, 0)),
                  pl.BlockSpec((NC, BN, DH), lambda i: (0, i, 0)),
                  pl.BlockSpec((BN, CW), lambda i: (i, 0)),
                  pl.BlockSpec((1, D), lambda i: (0, 0)),
                  pl.BlockSpec((D, D), lambda i: (0, 0))],
        out_specs=pl.BlockSpec((NC, BN, DH), lambda i: (0, i, 0)),
        out_shape=jax.ShapeDtypeStruct((NC, N, DH), jnp.float32),
    )(acc, h2, dinv, b_prev, W)


def _epilogue_last(acc, h2, dinv, b):
    def body(a_r, h_r, dv_r, b_r, o):
        dv = dv_r[...]
        oa = dv * (a_r[0] + h_r[0]) + b_r[:, :DH]
        ob = dv * (a_r[1] + h_r[1]) + b_r[:, DH:]
        o[...] = jnp.concatenate([oa, ob], axis=1)

    return pl.pallas_call(
        body,
        grid=(N // BN,),
        in_specs=[pl.BlockSpec((NC, BN, DH), lambda i: (0, i, 0)),
                  pl.BlockSpec((NC, BN, DH), lambda i: (0, i, 0)),
                  pl.BlockSpec((BN, CW), lambda i: (i, 0)),
                  pl.BlockSpec((1, D), lambda i: (0, 0))],
        out_specs=pl.BlockSpec((BN, D), lambda i: (i, 0)),
        out_shape=jax.ShapeDtypeStruct((N, D), jnp.float32),
    )(acc, h2, dinv, b)


def kernel(x, edge_index, W0, b0, W1, b1, W2, b2, W3, b3, W4, b4):
    src = edge_index[0]
    dst = edge_index[1]
    sidx = jnp.concatenate([src, src + N])
    Ws = [W0, W1, W2, W3, W4]
    bs = [b.reshape(1, D) for b in (b0, b1, b2, b3, b4)]

    cnt = _deg_counts(dst)
    h2, dinv = _matmul_first(x, Ws[0], cnt)
    for i in range(1, NUM_LAYERS):
        acc = _aggregate(h2.reshape(NC * N, DH), sidx, dst)
        h2 = _matmul_mid(acc, h2, dinv, bs[i - 1], Ws[i])
    acc = _aggregate(h2.reshape(NC * N, DH), sidx, dst)
    return _epilogue_last(acc, h2, dinv, bs[-1])


# speedup vs baseline: 1.0938x; 1.0058x over previous
import functools

import jax
import jax.numpy as jnp
from jax import lax
from jax.experimental import pallas as pl
from jax.experimental.pallas import tpu as pltpu
from jax.experimental.pallas import tpu_sc as plsc

N = 10000
D = 256
DH = 128
E = 160000
NUM_LAYERS = 5
NC = 2
NS = 16

CHUNK = 80
EP = E
EPS = EP // NS
NCHUNK = EPS // CHUNK
NP = 10240
RPS = NP // NS
ZC = 64

EPC = E // NC
DPS = EPC // NS
DCHUNK = 40
DNCHUNK = DPS // DCHUNK
CW = 128

BN = 2000

_MESH = plsc.VectorSubcoreMesh(core_axis_name="c", subcore_axis_name="s")


def _deg_counts(dst):
    ones = jnp.ones((DCHUNK, CW), jnp.float32)
    zeros = jnp.zeros((RPS, CW), jnp.float32)

    @functools.partial(
        pl.kernel,
        mesh=_MESH,
        out_type=jax.ShapeDtypeStruct((NC, NP, CW), jnp.float32),
        scratch_types=[
            pltpu.VMEM((DCHUNK,), jnp.int32),
            pltpu.VMEM((DCHUNK, CW), jnp.float32),
            pltpu.VMEM_SHARED((NP, CW), jnp.float32),
            pltpu.SemaphoreType.DMA,
        ],
    )
    def k(dst_hbm, ones_hbm, zeros_hbm, cnt_hbm, idx_v, ones_v, acc_sh, sem):
        c = lax.axis_index("c")
        s = lax.axis_index("s")
        row = pl.ds(s * RPS, RPS)
        pltpu.sync_copy(zeros_hbm, acc_sh.at[row])
        pltpu.sync_copy(ones_hbm, ones_v)
        plsc.subcore_barrier()
        base = c * EPC + s * DPS

        @pl.loop(0, DNCHUNK)
        def _(g):
            pltpu.sync_copy(dst_hbm.at[pl.ds(base + g * DCHUNK, DCHUNK)],
                            idx_v)
            pltpu.sync_copy(ones_v, acc_sh.at[idx_v], add=True)

        plsc.subcore_barrier()
        pltpu.sync_copy(acc_sh.at[row], cnt_hbm.at[c, row])

    return k(dst, ones, zeros)


def _aggregate(h2s, sidx, dst):
    zeros = jnp.zeros((RPS, DH), jnp.float32)

    @functools.partial(
        pl.kernel,
        mesh=_MESH,
        out_type=jax.ShapeDtypeStruct((NC, NP, DH), jnp.float32),
        scratch_types=[
            pltpu.VMEM((CHUNK,), jnp.int32),
            pltpu.VMEM((CHUNK,), jnp.int32),
            pltpu.VMEM((CHUNK, DH), jnp.float32),
            pltpu.VMEM_SHARED((NP, DH), jnp.float32),
            pltpu.SemaphoreType.DMA,
        ],
    )
    def k(h2s_hbm, sidx_hbm, dst_hbm, zeros_hbm, out_hbm,
          src_v, dst_v, rows_v, acc_sh, sem):
        c = lax.axis_index("c")
        s = lax.axis_index("s")
        row = pl.ds(s * RPS, RPS)
        pltpu.sync_copy(zeros_hbm, acc_sh.at[row])
        plsc.subcore_barrier()
        base = c * E + s * EPS

        @pl.loop(0, NCHUNK)
        def _(g):
            off = base + g * CHUNK
            pltpu.sync_copy(sidx_hbm.at[pl.ds(off, CHUNK)], src_v)
            pltpu.sync_copy(dst_hbm.at[pl.ds(s * EPS + g * CHUNK, CHUNK)],
                            dst_v)
            pltpu.async_copy(h2s_hbm.at[src_v], rows_v, sem).wait()
            pltpu.sync_copy(rows_v, acc_sh.at[dst_v], add=True)

        plsc.subcore_barrier()
        pltpu.sync_copy(acc_sh.at[row], out_hbm.at[c, row])

    return k(h2s, sidx, dst, zeros)


def _matmul_first(x, W, cnt):
    def body(x_r, w_r, c_r, o, dv_o):
        h = lax.dot_general(x_r[...], w_r[...], (((1,), (0,)), ((), ())),
                            precision=lax.Precision.HIGHEST,
                            preferred_element_type=jnp.float32)
        dv = lax.rsqrt(c_r[0] + c_r[1] + 1.0)
        dv_o[...] = dv
        o[0] = h[:, :DH] * dv
        o[1] = h[:, DH:] * dv

    return pl.pallas_call(
        body,
        grid=(N // BN,),
        in_specs=[pl.BlockSpec((BN, D), lambda i: (i, 0)),
                  pl.BlockSpec((D, D), lambda i: (0, 0)),
                  pl.BlockSpec((NC, BN, CW), lambda i: (0, i, 0))],
        out_specs=[pl.BlockSpec((NC, BN, DH), lambda i: (0, i, 0)),
                   pl.BlockSpec((BN, CW), lambda i: (i, 0))],
        out_shape=[jax.ShapeDtypeStruct((NC, N, DH), jnp.float32),
                   jax.ShapeDtypeStruct((N, CW), jnp.float32)],
    )(x, W, cnt)


def _matmul_mid(acc, h2, dinv, b_prev, W):
    def body(a_r, h_r, dv_r, b_r, w_r, o):
        dv = dv_r[...]
        za = dv * (a_r[0] + h_r[0]) + b_r[:, :DH]
        zb = dv * (a_r[1] + h_r[1]) + b_r[:, DH:]
        z = jnp.maximum(jnp.concatenate([za, zb], axis=1), 0.0)
        h = lax.dot_general(z, w_r[...], (((1,), (0,)), ((), ())),
                            precision=lax.Precision.HIGHEST,
                            preferred_element_type=jnp.float32)
        o[0] = h[:, :DH] * dv
        o[1] = h[:, DH:] * dv

    return pl.pallas_call(
        body,
        grid=(N // BN,),
        in_specs=[pl.BlockSpec((NC, BN, DH), lambda i: (0, i, 0)),
                  pl.BlockSpec((NC, BN, DH), lambda i: (0, i, 0)),
                  pl.BlockSpec((BN, CW), lambda i: (i, 0)),
                  pl.BlockSpec((1, D), lambda i: (0, 0)),
                  pl.BlockSpec((D, D), lambda i: (0, 0))],
        out_specs=pl.BlockSpec((NC, BN, DH), lambda i: (0, i, 0)),
        out_shape=jax.ShapeDtypeStruct((NC, N, DH), jnp.float32),
    )(acc, h2, dinv, b_prev, W)


def _epilogue_last(acc, h2, dinv, b):
    def body(a_r, h_r, dv_r, b_r, o):
        dv = dv_r[...]
        oa = dv * (a_r[0] + h_r[0]) + b_r[:, :DH]
        ob = dv * (a_r[1] + h_r[1]) + b_r[:, DH:]
        o[...] = jnp.concatenate([oa, ob], axis=1)

    return pl.pallas_call(
        body,
        grid=(N // BN,),
        in_specs=[pl.BlockSpec((NC, BN, DH), lambda i: (0, i, 0)),
                  pl.BlockSpec((NC, BN, DH), lambda i: (0, i, 0)),
                  pl.BlockSpec((BN, CW), lambda i: (i, 0)),
                  pl.BlockSpec((1, D), lambda i: (0, 0))],
        out_specs=pl.BlockSpec((BN, D), lambda i: (i, 0)),
        out_shape=jax.ShapeDtypeStruct((N, D), jnp.float32),
    )(acc, h2, dinv, b)


def kernel(x, edge_index, W0, b0, W1, b1, W2, b2, W3, b3, W4, b4):
    src = edge_index[0]
    dst = edge_index[1]
    sidx = jnp.concatenate([src, src + N])
    Ws = [W0, W1, W2, W3, W4]
    bs = [b.reshape(1, D) for b in (b0, b1, b2, b3, b4)]

    cnt = _deg_counts(dst)
    h2, dinv = _matmul_first(x, Ws[0], cnt)
    for i in range(1, NUM_LAYERS):
        acc = _aggregate(h2.reshape(NC * N, DH), sidx, dst)
        h2 = _matmul_mid(acc, h2, dinv, bs[i - 1], Ws[i])
    acc = _aggregate(h2.reshape(NC * N, DH), sidx, dst)
    return _epilogue_last(acc, h2, dinv, bs[-1])
